# baseline (speedup 1.0000x reference)
import functools

import jax
import jax.numpy as jnp
import numpy as np
from jax import lax
from jax.experimental import pallas as pl
from jax.experimental.pallas import tpu as pltpu
from jax.experimental.pallas import tpu_sc as plsc

NUM_TOPICS = 32
H = 128
EPS = 1e-5
B = 4096
L = 200
C0, C1 = 104, 96
NC, NS = 2, 16
NW = NC * NS
BPW = B // NW
HW = H // 2
GV = HW // 16

_PERM = np.concatenate(
    [np.concatenate([32 * g + 2 * np.arange(16), 32 * g + 2 * np.arange(16) + 1])
     for g in range(H // 32)]
).astype(np.int32)


def _pool_sc(table_bf16, idx):
    mesh = plsc.VectorSubcoreMesh(core_axis_name="c", subcore_axis_name="s")

    @functools.partial(
        pl.kernel,
        out_type=jax.ShapeDtypeStruct((B, H), jnp.float32),
        mesh=mesh,
        scratch_types=[
            pltpu.VMEM((BPW, L), jnp.int32),
            pltpu.VMEM((4, 2, C0, H), jnp.bfloat16),
            pltpu.VMEM((BPW, H), jnp.float32),
            pltpu.SemaphoreType.DMA,
            pltpu.SemaphoreType.DMA,
            pltpu.SemaphoreType.DMA,
            pltpu.SemaphoreType.DMA,
        ],
        compiler_params=pltpu.CompilerParams(
            use_tc_tiling_on_sc=False, needs_layout_passes=False),
    )
    def pool(table_hbm, idx_hbm, out_hbm, idx_v, rows_v, out_v,
             sem0, sem1, sem2, sem3):
        wid = lax.axis_index("s") * NC + lax.axis_index("c")
        base = wid * BPW
        pltpu.sync_copy(idx_hbm.at[pl.ds(base, BPW)], idx_v)
        sems = (sem0, sem1, sem2, sem3)

        def fire(b, p):
            pltpu.async_copy(
                table_hbm.at[idx_v.at[b, pl.ds(0, C0)]],
                rows_v.at[p, 0],
                sems[p],
            )
            pltpu.async_copy(
                table_hbm.at[idx_v.at[b, pl.ds(C0, C1)]],
                rows_v.at[p, 1, pl.ds(0, C1)],
                sems[p],
            )

        def drain(p):
            pltpu.make_async_copy(
                table_hbm.at[pl.ds(0, C0)], rows_v.at[p, 0], sems[p]
            ).wait()
            pltpu.make_async_copy(
                table_hbm.at[pl.ds(0, C1)], rows_v.at[p, 1, pl.ds(0, C1)],
                sems[p],
            ).wait()

        def process(b, p):
            drain(p)

            def make_body(j):
                def l_body(m, accs):
                    out = list(accs)
                    for g in range(GV):
                        t = [rows_v[p, j, 8 * m + r, pl.ds(32 * g, 32)]
                             for r in range(8)]
                        t = [t[0] + t[1], t[2] + t[3], t[4] + t[5], t[6] + t[7]]
                        t = [t[0] + t[1], t[2] + t[3]]
                        w = t[0] + t[1]
                        lo, hi = plsc.unpack(w, format=plsc.PackFormat.INTERLEAVED)
                        out[2 * g] = out[2 * g] + lo
                        out[2 * g + 1] = out[2 * g + 1] + hi
                    return tuple(out)
                return l_body

            accs = tuple(jnp.zeros((16,), jnp.float32) for _ in range(2 * GV))
            accs = lax.fori_loop(0, C0 // 8, make_body(0), accs)
            accs = lax.fori_loop(0, C1 // 8, make_body(1), accs)
            for g in range(GV):
                out_v[b, pl.ds(32 * g, 16)] = accs[2 * g]
                out_v[b, pl.ds(32 * g + 16, 16)] = accs[2 * g + 1]

        fire(0, 0)
        fire(1, 1)
        fire(2, 2)

        def body(i, _):
            for q in range(4):
                b = i * 4 + q
                fire(b + 3, (q + 3) % 4)
                process(b, q)
            return 0

        lax.fori_loop(0, (BPW - 4) // 4, body, 0)
        fire(BPW - 1, (BPW - 1) % 4)
        process(BPW - 4, (BPW - 4) % 4)
        process(BPW - 3, (BPW - 3) % 4)
        process(BPW - 2, (BPW - 2) % 4)
        process(BPW - 1, (BPW - 1) % 4)
        pltpu.sync_copy(out_v, out_hbm.at[pl.ds(base, BPW)])

    return pool(table_bf16, idx)


_CAST_BLK = 2048


def _cast_tc(table):
    v = table.shape[0]
    grid = (v + _CAST_BLK - 1) // _CAST_BLK

    def body(s_ref, o_ref):
        o_ref[...] = s_ref[...].astype(jnp.bfloat16)

    return pl.pallas_call(
        body,
        grid=(grid,),
        in_specs=[pl.BlockSpec((_CAST_BLK, H), lambda i: (i, 0))],
        out_specs=pl.BlockSpec((_CAST_BLK, H), lambda i: (i, 0)),
        out_shape=jax.ShapeDtypeStruct((v, H), jnp.bfloat16),
    )(table)


def _mlp_tc(S, idx, read_depth, row0p, W1a, w1b, b1, g1, be1, W2, b2, g2, be2):
    inv = float(1.0 / (1.0 + EPS) ** 0.5)

    def body(s_ref, idx_ref, rd_ref, row0_ref, w1a_ref, w1b_ref, b1_ref, g1_ref,
             be1_ref, w2_ref, b2_ref, g2_ref, be2_ref, loc_ref, scale_ref):
        rd = rd_ref[...]
        n0 = jnp.sum((idx_ref[...] == 0).astype(jnp.float32), axis=1,
                     keepdims=True)
        ave = (s_ref[...] - n0 * row0_ref[...]) / rd
        h = jnp.dot(ave, w1a_ref[...], preferred_element_type=jnp.float32)
        h = h + jnp.log(rd) * w1b_ref[...] + b1_ref[...]
        h = g1_ref[...] * h * inv + be1_ref[...]
        h = jnp.maximum(h, 0.0)
        o = jnp.dot(h, w2_ref[...], preferred_element_type=jnp.float32)
        o = o + b2_ref[...]
        o = g2_ref[...] * o * inv + be2_ref[...]
        loc_ref[...] = o[:, :NUM_TOPICS]
        x = o[:, NUM_TOPICS:]
        scale_ref[...] = jnp.maximum(x, 0.0) + jnp.log1p(jnp.exp(-jnp.abs(x)))

    return pl.pallas_call(
        body,
        out_shape=(
            jax.ShapeDtypeStruct((B, NUM_TOPICS), jnp.float32),
            jax.ShapeDtypeStruct((B, NUM_TOPICS), jnp.float32),
        ),
    )(S, idx, read_depth, row0p, W1a, w1b, b1, g1, be1, W2, b2, g2, be2)


def kernel(idx, read_depth, emb_table, W1, b1, g1, be1, W2, b2, g2, be2):
    idx = idx.astype(jnp.int32)
    table_bf16 = _cast_tc(emb_table)
    S = _pool_sc(table_bf16, idx)
    perm = jnp.asarray(_PERM)
    row0p = table_bf16[0, perm].astype(jnp.float32).reshape(1, H)
    W1a = W1[:H, :][perm, :]
    w1b = W1[H:, :]
    return _mlp_tc(
        S, idx, read_depth, row0p, W1a, w1b,
        b1.reshape(1, H), g1.reshape(1, H), be1.reshape(1, H),
        W2, b2.reshape(1, 2 * NUM_TOPICS),
        g2.reshape(1, 2 * NUM_TOPICS), be2.reshape(1, 2 * NUM_TOPICS),
    )


# speedup vs baseline: 1.5703x; 1.5703x over previous
import functools

import jax
import jax.numpy as jnp
import numpy as np
from jax import lax
from jax.experimental import pallas as pl
from jax.experimental.pallas import tpu as pltpu
from jax.experimental.pallas import tpu_sc as plsc

NUM_TOPICS = 32
H = 128
EPS = 1e-5
B = 4096
L = 200
C0, C1 = 104, 96
NC, NS = 2, 16
NW = NC * NS
BPW = B // NW
HW = H // 2
GV = HW // 16

V = 100001
RPT = 3125
_CB = 256
_CBS = [_CB] * (RPT // _CB) + [RPT % _CB]


def _cast_sc(table_f32):
    mesh = plsc.VectorSubcoreMesh(core_axis_name="c", subcore_axis_name="s")

    @functools.partial(
        pl.kernel,
        out_type=jax.ShapeDtypeStruct((V, H), jnp.bfloat16),
        mesh=mesh,
        scratch_types=[
            pltpu.VMEM((2, _CB, H), jnp.float32),
            pltpu.VMEM((2, _CB, H), jnp.bfloat16),
            pltpu.SemaphoreType.DMA,
            pltpu.SemaphoreType.DMA,
            pltpu.SemaphoreType.DMA,
            pltpu.SemaphoreType.DMA,
        ],
        compiler_params=pltpu.CompilerParams(
            use_tc_tiling_on_sc=False, needs_layout_passes=False),
    )
    def cast(tf_hbm, out_hbm, inb, outb, isem0, isem1, osem0, osem1):
        wid = lax.axis_index("s") * NC + lax.axis_index("c")
        base = wid * RPT
        isems = (isem0, isem1)
        osems = (osem0, osem1)
        nblk = len(_CBS)

        def fire_in(kb):
            pltpu.async_copy(
                tf_hbm.at[pl.ds(base + kb * _CB, _CBS[kb])],
                inb.at[kb % 2, pl.ds(0, _CBS[kb])],
                isems[kb % 2],
            )

        fire_in(0)
        for kb in range(nblk):
            p = kb % 2
            if kb + 1 < nblk:
                fire_in(kb + 1)
            pltpu.make_async_copy(
                tf_hbm.at[pl.ds(0, _CBS[kb])],
                inb.at[p, pl.ds(0, _CBS[kb])],
                isems[p],
            ).wait()
            if kb >= 2:
                pltpu.make_async_copy(
                    tf_hbm.at[pl.ds(0, _CBS[kb - 2])],
                    outb.at[p, pl.ds(0, _CBS[kb - 2])],
                    osems[p],
                ).wait()

            def r_body(r, _):
                for g in range(GV):
                    v0 = inb[p, r, pl.ds(32 * g, 16)]
                    v1 = inb[p, r, pl.ds(32 * g + 16, 16)]
                    w = plsc.pack(v0, v1, format=plsc.PackFormat.INTERLEAVED)
                    outb[p, r, pl.ds(32 * g, 32)] = w
                return 0

            lax.fori_loop(0, _CBS[kb], r_body, 0)
            pltpu.async_copy(
                outb.at[p, pl.ds(0, _CBS[kb])],
                out_hbm.at[pl.ds(base + kb * _CB, _CBS[kb])],
                osems[p],
            )
        for kb in (nblk - 2, nblk - 1):
            pltpu.make_async_copy(
                tf_hbm.at[pl.ds(0, _CBS[kb])],
                outb.at[kb % 2, pl.ds(0, _CBS[kb])],
                osems[kb % 2],
            ).wait()

        @pl.when(wid == 0)
        def _():
            pltpu.sync_copy(tf_hbm.at[pl.ds(V - 1, 1)], inb.at[0, pl.ds(0, 1)])

            def r1_body(r, _):
                for g in range(GV):
                    v0 = inb[0, r, pl.ds(32 * g, 16)]
                    v1 = inb[0, r, pl.ds(32 * g + 16, 16)]
                    outb[0, r, pl.ds(32 * g, 32)] = plsc.pack(
                        v0, v1, format=plsc.PackFormat.INTERLEAVED)
                return 0

            lax.fori_loop(0, 1, r1_body, 0)
            pltpu.sync_copy(outb.at[0, pl.ds(0, 1)], out_hbm.at[pl.ds(V - 1, 1)])

    return cast(table_f32)


def _pool_sc(table_bf16, idx):
    mesh = plsc.VectorSubcoreMesh(core_axis_name="c", subcore_axis_name="s")

    @functools.partial(
        pl.kernel,
        out_type=jax.ShapeDtypeStruct((B, H), jnp.float32),
        mesh=mesh,
        scratch_types=[
            pltpu.VMEM((BPW, L), jnp.int32),
            pltpu.VMEM((4, 2, C0, H), jnp.bfloat16),
            pltpu.VMEM((BPW, H), jnp.float32),
            pltpu.SemaphoreType.DMA,
            pltpu.SemaphoreType.DMA,
            pltpu.SemaphoreType.DMA,
            pltpu.SemaphoreType.DMA,
        ],
        compiler_params=pltpu.CompilerParams(
            use_tc_tiling_on_sc=False, needs_layout_passes=False),
    )
    def pool(table_hbm, idx_hbm, out_hbm, idx_v, rows_v, out_v,
             sem0, sem1, sem2, sem3):
        wid = lax.axis_index("s") * NC + lax.axis_index("c")
        base = wid * BPW
        pltpu.sync_copy(idx_hbm.at[pl.ds(base, BPW)], idx_v)
        sems = (sem0, sem1, sem2, sem3)

        def fire(b, p):
            pltpu.async_copy(
                table_hbm.at[idx_v.at[b, pl.ds(0, C0)]],
                rows_v.at[p, 0],
                sems[p],
            )
            pltpu.async_copy(
                table_hbm.at[idx_v.at[b, pl.ds(C0, C1)]],
                rows_v.at[p, 1, pl.ds(0, C1)],
                sems[p],
            )

        def drain(p):
            pltpu.make_async_copy(
                table_hbm.at[pl.ds(0, C0)], rows_v.at[p, 0], sems[p]
            ).wait()
            pltpu.make_async_copy(
                table_hbm.at[pl.ds(0, C1)], rows_v.at[p, 1, pl.ds(0, C1)],
                sems[p],
            ).wait()

        def process(b, p):
            drain(p)

            def make_body(j):
                def l_body(m, accs):
                    out = list(accs)
                    for g in range(GV):
                        t = [rows_v[p, j, 8 * m + r, pl.ds(32 * g, 32)]
                             for r in range(8)]
                        t = [t[0] + t[1], t[2] + t[3], t[4] + t[5], t[6] + t[7]]
                        t = [t[0] + t[1], t[2] + t[3]]
                        w = t[0] + t[1]
                        lo, hi = plsc.unpack(w, format=plsc.PackFormat.INTERLEAVED)
                        out[2 * g] = out[2 * g] + lo
                        out[2 * g + 1] = out[2 * g + 1] + hi
                    return tuple(out)
                return l_body

            accs = tuple(jnp.zeros((16,), jnp.float32) for _ in range(2 * GV))
            accs = lax.fori_loop(0, C0 // 8, make_body(0), accs)
            accs = lax.fori_loop(0, C1 // 8, make_body(1), accs)
            for g in range(GV):
                out_v[b, pl.ds(32 * g, 16)] = accs[2 * g]
                out_v[b, pl.ds(32 * g + 16, 16)] = accs[2 * g + 1]

        fire(0, 0)
        fire(1, 1)
        fire(2, 2)

        def body(i, _):
            for q in range(4):
                b = i * 4 + q
                fire(b + 3, (q + 3) % 4)
                process(b, q)
            return 0

        lax.fori_loop(0, (BPW - 4) // 4, body, 0)
        fire(BPW - 1, (BPW - 1) % 4)
        process(BPW - 4, (BPW - 4) % 4)
        process(BPW - 3, (BPW - 3) % 4)
        process(BPW - 2, (BPW - 2) % 4)
        process(BPW - 1, (BPW - 1) % 4)
        pltpu.sync_copy(out_v, out_hbm.at[pl.ds(base, BPW)])

    return pool(table_bf16, idx)


def _mlp_tc(S, idx, read_depth, row0p, W1a, w1b, b1, g1, be1, W2, b2, g2, be2):
    inv = float(1.0 / (1.0 + EPS) ** 0.5)

    def body(s_ref, idx_ref, rd_ref, row0_ref, w1a_ref, w1b_ref, b1_ref, g1_ref,
             be1_ref, w2_ref, b2_ref, g2_ref, be2_ref, loc_ref, scale_ref):
        rd = rd_ref[...]
        n0 = jnp.sum((idx_ref[...] == 0).astype(jnp.float32), axis=1,
                     keepdims=True)
        ave = (s_ref[...] - n0 * row0_ref[...]) / rd
        h = jnp.dot(ave, w1a_ref[...], preferred_element_type=jnp.float32)
        h = h + jnp.log(rd) * w1b_ref[...] + b1_ref[...]
        h = g1_ref[...] * h * inv + be1_ref[...]
        h = jnp.maximum(h, 0.0)
        o = jnp.dot(h, w2_ref[...], preferred_element_type=jnp.float32)
        o = o + b2_ref[...]
        o = g2_ref[...] * o * inv + be2_ref[...]
        loc_ref[...] = o[:, :NUM_TOPICS]
        x = o[:, NUM_TOPICS:]
        scale_ref[...] = jnp.maximum(x, 0.0) + jnp.log1p(jnp.exp(-jnp.abs(x)))

    return pl.pallas_call(
        body,
        out_shape=(
            jax.ShapeDtypeStruct((B, NUM_TOPICS), jnp.float32),
            jax.ShapeDtypeStruct((B, NUM_TOPICS), jnp.float32),
        ),
    )(S, idx, read_depth, row0p, W1a, w1b, b1, g1, be1, W2, b2, g2, be2)


def kernel(idx, read_depth, emb_table, W1, b1, g1, be1, W2, b2, g2, be2):
    idx = idx.astype(jnp.int32)
    table_bf16 = _cast_sc(emb_table)
    S = _pool_sc(table_bf16, idx)
    row0p = emb_table[0:1, :].astype(jnp.bfloat16).astype(jnp.float32)
    W1a = W1[:H, :]
    w1b = W1[H:, :]
    return _mlp_tc(
        S, idx, read_depth, row0p, W1a, w1b,
        b1.reshape(1, H), g1.reshape(1, H), be1.reshape(1, H),
        W2, b2.reshape(1, 2 * NUM_TOPICS),
        g2.reshape(1, 2 * NUM_TOPICS), be2.reshape(1, 2 * NUM_TOPICS),
    )
